# baseline (speedup 1.0000x reference)
import functools

import jax
import jax.numpy as jnp
from jax import lax
from jax.experimental import pallas as pl
from jax.experimental.pallas import tpu as pltpu
from jax.experimental.pallas import tpu_sc as plsc

_N_PATH = 1_000_000
_PATH_DIM = 3
_CONTROL_DIM = 4

_NC = 2
_NS = 16
_NW = _NC * _NS
_L = 16

_PTS_W = 31_248
_FLOATS_W = _PTS_W * 3
_GROUPS_W = _PTS_W // _L
_UNROLL = 3
_ITERS = _GROUPS_W // _UNROLL

_TAIL_PTS = _N_PATH - _NW * _PTS_W
_TAIL_GROUPS = _TAIL_PTS // _L
_TAIL_START = _NW * _PTS_W
_TAIL_FLOAT_OFF = _TAIL_START * 3

_BIG = float(jnp.finfo(jnp.float32).max)
_IMAX = 2**31 - 1


def _merge(m0, a0, m1, a1):
    take1 = (m1 < m0) | ((m1 == m0) & (a1 < a0))
    return jnp.where(take1, m1, m0), jnp.where(take1, a1, a0)


def _sc_closest_point(path_flat, svec):
    mesh = plsc.VectorSubcoreMesh(core_axis_name="c", subcore_axis_name="s")

    @functools.partial(
        pl.kernel,
        mesh=mesh,
        compiler_params=pltpu.CompilerParams(needs_layout_passes=False),
        out_type=[
            jax.ShapeDtypeStruct((_L,), jnp.float32),
            jax.ShapeDtypeStruct((_NC, _L), jnp.float32),
            jax.ShapeDtypeStruct((_NC, _L), jnp.int32),
        ],
        scratch_types=[
            pltpu.VMEM((_FLOATS_W,), jnp.float32),
            pltpu.VMEM((_TAIL_PTS * 3,), jnp.float32),
            pltpu.VMEM((3, _L), jnp.float32),
            pltpu.VMEM((_L,), jnp.float32),
            pltpu.VMEM((_L,), jnp.int32),
            pltpu.VMEM((_NS, _L), jnp.float32),
            pltpu.VMEM((_NS, _L), jnp.int32),
            pltpu.VMEM_SHARED((_NS, _L), jnp.float32),
            pltpu.VMEM_SHARED((_NS, _L), jnp.int32),
        ],
    )
    def k(path_hbm, svec_hbm, ctrl_hbm, outd_hbm, outi_hbm,
          buf, tailbuf, svec_v, mbuf, abuf, allm_v, alli_v, shm_f, shm_i):
        c = lax.axis_index("c")
        s = lax.axis_index("s")
        wid = s * _NC + c

        pltpu.sync_copy(svec_hbm, svec_v)
        pltpu.sync_copy(path_hbm.at[pl.ds(wid * _FLOATS_W, _FLOATS_W)], buf)
        sx = svec_v[0]
        sy = svec_v[1]
        sz = svec_v[2]

        iota = lax.iota(jnp.int32, _L)
        xi0 = iota * 3
        gi0 = wid * _PTS_W + iota

        def group_update(xij, gij, m, a):
            x = plsc.load_gather(buf, [xij])
            y = plsc.load_gather(buf, [xij + 1])
            z = plsc.load_gather(buf, [xij + 2])
            dx = x - sx
            dy = y - sy
            dz = z - sz
            d2 = dx * dx + dy * dy + dz * dz
            take = d2 < m
            return jnp.where(take, d2, m), jnp.where(take, gij, a)

        def body(_, carry):
            m0, a0, m1, a1, m2, a2, xi, gi = carry
            m0, a0 = group_update(xi, gi, m0, a0)
            m1, a1 = group_update(xi + 48, gi + 16, m1, a1)
            m2, a2 = group_update(xi + 96, gi + 32, m2, a2)
            return (m0, a0, m1, a1, m2, a2, xi + 48 * _UNROLL, gi + 16 * _UNROLL)

        big = jnp.full((_L,), _BIG, jnp.float32)
        zero_i = jnp.zeros((_L,), jnp.int32)
        init = (big, zero_i, big, zero_i, big, zero_i, xi0, gi0)
        m0, a0, m1, a1, m2, a2, _, _ = lax.fori_loop(0, _ITERS, body, init)
        m, a = _merge(m0, a0, m1, a1)
        m, a = _merge(m, a, m2, a2)

        def group_update2(xij, gij, m_, a_):
            x = plsc.load_gather(tailbuf, [xij])
            y = plsc.load_gather(tailbuf, [xij + 1])
            z = plsc.load_gather(tailbuf, [xij + 2])
            dx = x - sx
            dy = y - sy
            dz = z - sz
            d2 = dx * dx + dy * dy + dz * dz
            take = d2 < m_
            return jnp.where(take, d2, m_), jnp.where(take, gij, a_)

        @pl.when(wid == _NW - 1)
        def _():
            pltpu.sync_copy(
                path_hbm.at[pl.ds(_TAIL_FLOAT_OFF, _TAIL_PTS * 3)], tailbuf)
            tm, ta = m, a
            for g in range(_TAIL_GROUPS):
                tm, ta = group_update2(xi0 + 48 * g, _TAIL_START + 16 * g + iota,
                                       tm, ta)
            mbuf[...] = tm
            abuf[...] = ta

        @pl.when(wid != _NW - 1)
        def _():
            mbuf[...] = m
            abuf[...] = a

        pltpu.sync_copy(mbuf, shm_f.at[s])
        pltpu.sync_copy(abuf, shm_i.at[s])
        plsc.subcore_barrier()

        @pl.when(s == 0)
        def _():
            pltpu.sync_copy(shm_f, allm_v)
            pltpu.sync_copy(shm_i, alli_v)
            gm = allm_v[0]
            ga = alli_v[0]
            for r in range(1, _NS):
                gm, ga = _merge(gm, ga, allm_v[r], alli_v[r])
            best = jnp.min(gm)
            cand = jnp.where(gm == jnp.full((_L,), best, jnp.float32),
                             ga, jnp.full((_L,), _IMAX, jnp.int32))
            besti = jnp.min(cand)
            mbuf[...] = jnp.full((_L,), best, jnp.float32)
            abuf[...] = jnp.full((_L,), besti, jnp.int32)
            pltpu.sync_copy(mbuf, outd_hbm.at[c])
            pltpu.sync_copy(abuf, outi_hbm.at[c])

            @pl.when(c == 0)
            def _():
                mbuf[...] = jnp.zeros((_L,), jnp.float32)
                pltpu.sync_copy(mbuf, ctrl_hbm)

    return k(path_flat, svec)


def kernel(state, path):
    path_flat = path.reshape(-1)
    svec = jnp.broadcast_to(state[0, :_PATH_DIM][:, None], (3, _L))
    ctrl16, _best_d2, _best_idx = _sc_closest_point(path_flat, svec)
    return ctrl16[:_CONTROL_DIM].reshape(1, _CONTROL_DIM)


# speedup vs baseline: 6.4576x; 6.4576x over previous
import functools

import jax
import jax.numpy as jnp
from jax import lax
from jax.experimental import pallas as pl
from jax.experimental.pallas import tpu as pltpu
from jax.experimental.pallas import tpu_sc as plsc

_N_PATH = 1_000_000
_PATH_DIM = 3
_CONTROL_DIM = 4

_NC = 2
_NS = 16
_NW = _NC * _NS
_L = 16

_PTS_W = 31_248
_CH = 336
_NCHUNK = _PTS_W // _CH
_GROUPS_CH = _CH // _L

_TAIL_PTS = _N_PATH - _NW * _PTS_W
_TAIL_GROUPS = _TAIL_PTS // _L
_TAIL_START = _NW * _PTS_W

_BIG = float(jnp.finfo(jnp.float32).max)
_IMAX = 2**31 - 1


def _merge(m0, a0, m1, a1):
    take1 = (m1 < m0) | ((m1 == m0) & (a1 < a0))
    return jnp.where(take1, m1, m0), jnp.where(take1, a1, a0)


def _sc_closest_point(path, svec):
    mesh = plsc.VectorSubcoreMesh(core_axis_name="c", subcore_axis_name="s")

    @functools.partial(
        pl.kernel,
        mesh=mesh,
        compiler_params=pltpu.CompilerParams(needs_layout_passes=False),
        out_type=[
            jax.ShapeDtypeStruct((_L,), jnp.float32),
            jax.ShapeDtypeStruct((_NC, _L), jnp.float32),
            jax.ShapeDtypeStruct((_NC, _L), jnp.int32),
            jax.ShapeDtypeStruct((_NC, _NS, _L), jnp.float32),
            jax.ShapeDtypeStruct((_NC, _NS, _L), jnp.int32),
        ],
        scratch_types=[
            pltpu.VMEM((_CH, _PATH_DIM), jnp.float32),
            pltpu.VMEM((_TAIL_PTS, _PATH_DIM), jnp.float32),
            pltpu.VMEM((3, _L), jnp.float32),
            pltpu.VMEM((_L,), jnp.float32),
            pltpu.VMEM((_L,), jnp.int32),
            pltpu.VMEM((_NS, _L), jnp.float32),
            pltpu.VMEM((_NS, _L), jnp.int32),
        ],
    )
    def k(path_hbm, svec_hbm, ctrl_hbm, outd_hbm, outi_hbm, partm_hbm, parta_hbm,
          buf, tailbuf, svec_v, mbuf, abuf, allm_v, alli_v):
        c = lax.axis_index("c")
        s = lax.axis_index("s")
        wid = s * _NC + c

        pltpu.sync_copy(svec_hbm, svec_v)
        sx = svec_v[0]
        sy = svec_v[1]
        sz = svec_v[2]

        iota = lax.iota(jnp.int32, _L)
        col0 = jnp.zeros((_L,), jnp.int32)
        col1 = col0 + 1
        col2 = col0 + 2

        def group_update(ref, pij, gij, m, a):
            x = plsc.load_gather(ref, [pij, col0])
            y = plsc.load_gather(ref, [pij, col1])
            z = plsc.load_gather(ref, [pij, col2])
            dx = x - sx
            dy = y - sy
            dz = z - sz
            d2 = dx * dx + dy * dy + dz * dz
            take = d2 < m
            return jnp.where(take, d2, m), jnp.where(take, gij, a)

        def chunk_body(t, carry):
            accs = list(carry)
            base = wid * _PTS_W + t * _CH
            pltpu.sync_copy(path_hbm.at[pl.ds(base, _CH)], buf)
            gbase = base + iota
            for g in range(_GROUPS_CH):
                j = g % 3
                m_, a_ = group_update(buf, iota + 16 * g, gbase + 16 * g,
                                      accs[2 * j], accs[2 * j + 1])
                accs[2 * j] = m_
                accs[2 * j + 1] = a_
            return tuple(accs)

        big = jnp.full((_L,), _BIG, jnp.float32)
        zero_i = jnp.zeros((_L,), jnp.int32)
        init = (big, zero_i, big, zero_i, big, zero_i)
        m0, a0, m1, a1, m2, a2 = lax.fori_loop(0, _NCHUNK, chunk_body, init)
        m, a = _merge(m0, a0, m1, a1)
        m, a = _merge(m, a, m2, a2)

        @pl.when(wid == _NW - 1)
        def _():
            pltpu.sync_copy(path_hbm.at[pl.ds(_TAIL_START, _TAIL_PTS)], tailbuf)
            tm, ta = m, a
            for g in range(_TAIL_GROUPS):
                tm, ta = group_update(tailbuf, iota + 16 * g,
                                      _TAIL_START + 16 * g + iota, tm, ta)
            mbuf[...] = tm
            abuf[...] = ta

        @pl.when(wid != _NW - 1)
        def _():
            mbuf[...] = m
            abuf[...] = a

        pltpu.sync_copy(mbuf, partm_hbm.at[c, s])
        pltpu.sync_copy(abuf, parta_hbm.at[c, s])
        plsc.subcore_barrier()

        @pl.when(s == 0)
        def _():
            pltpu.sync_copy(partm_hbm.at[c], allm_v)
            pltpu.sync_copy(parta_hbm.at[c], alli_v)
            gm = allm_v[0]
            ga = alli_v[0]
            for r in range(1, _NS):
                gm, ga = _merge(gm, ga, allm_v[r], alli_v[r])
            best = jnp.min(gm)
            cand = jnp.where(gm == jnp.full((_L,), best, jnp.float32),
                             ga, jnp.full((_L,), _IMAX, jnp.int32))
            besti = jnp.min(cand)
            mbuf[...] = jnp.full((_L,), best, jnp.float32)
            abuf[...] = jnp.full((_L,), besti, jnp.int32)
            pltpu.sync_copy(mbuf, outd_hbm.at[c])
            pltpu.sync_copy(abuf, outi_hbm.at[c])

            @pl.when(c == 0)
            def _():
                mbuf[...] = jnp.zeros((_L,), jnp.float32)
                pltpu.sync_copy(mbuf, ctrl_hbm)

    return k(path, svec)


def kernel(state, path):
    svec = jnp.broadcast_to(state[0, :_PATH_DIM][:, None], (3, _L))
    ctrl16, _best_d2, _best_idx, _dm, _da = _sc_closest_point(path, svec)
    return ctrl16[:_CONTROL_DIM].reshape(1, _CONTROL_DIM)


# speedup vs baseline: 71.2661x; 11.0360x over previous
import functools

import jax
import jax.numpy as jnp
from jax import lax
from jax.experimental import pallas as pl
from jax.experimental.pallas import tpu as pltpu
from jax.experimental.pallas import tpu_sc as plsc

_N_PATH = 1_000_000
_PATH_DIM = 3
_CONTROL_DIM = 4

_NC = 2
_NS = 16
_NW = _NC * _NS
_L = 16

_PTS_W = 31_360
_N_PAD = _NW * _PTS_W
_GROUPS_W = _PTS_W // _L
_UNROLL = 8
_ITERS = _GROUPS_W // _UNROLL

_BIG = float(jnp.finfo(jnp.float32).max)
_IMAX = 2**31 - 1


def _merge(m0, a0, m1, a1):
    take1 = (m1 < m0) | ((m1 == m0) & (a1 < a0))
    return jnp.where(take1, m1, m0), jnp.where(take1, a1, a0)


def _sc_closest_point(path_t, svec):
    mesh = plsc.VectorSubcoreMesh(core_axis_name="c", subcore_axis_name="s")

    @functools.partial(
        pl.kernel,
        mesh=mesh,
        compiler_params=pltpu.CompilerParams(needs_layout_passes=False),
        out_type=[
            jax.ShapeDtypeStruct((_L,), jnp.float32),
            jax.ShapeDtypeStruct((_NC, _L), jnp.float32),
            jax.ShapeDtypeStruct((_NC, _L), jnp.int32),
            jax.ShapeDtypeStruct((_NC, _NS, _L), jnp.float32),
            jax.ShapeDtypeStruct((_NC, _NS, _L), jnp.int32),
        ],
        scratch_types=[
            pltpu.VMEM((3, _PTS_W), jnp.float32),
            pltpu.VMEM((3, _L), jnp.float32),
            pltpu.VMEM((_L,), jnp.float32),
            pltpu.VMEM((_L,), jnp.int32),
            pltpu.VMEM((_NS, _L), jnp.float32),
            pltpu.VMEM((_NS, _L), jnp.int32),
            pltpu.SemaphoreType.DMA,
        ],
    )
    def k(path_hbm, svec_hbm, ctrl_hbm, outd_hbm, outi_hbm, partm_hbm, parta_hbm,
          buf, svec_v, mbuf, abuf, allm_v, alli_v, sem):
        c = lax.axis_index("c")
        s = lax.axis_index("s")
        wid = s * _NC + c
        p0 = pl.multiple_of(wid * _PTS_W, 128)

        pltpu.sync_copy(svec_hbm, svec_v)
        pltpu.async_copy(path_hbm.at[:, pl.ds(p0, _PTS_W)], buf, sem).wait()

        sx = svec_v[0]
        sy = svec_v[1]
        sz = svec_v[2]
        iota = lax.iota(jnp.int32, _L)

        def group_update(b, off, gij, m, a):
            x = b[0, pl.ds(off, _L)]
            y = b[1, pl.ds(off, _L)]
            z = b[2, pl.ds(off, _L)]
            dx = x - sx
            dy = y - sy
            dz = z - sz
            d2 = dx * dx + dy * dy + dz * dz
            take = d2 < m
            return jnp.where(take, d2, m), jnp.where(take, gij, a)

        def body(t, carry):
            accs = list(carry[:-1])
            gi = carry[-1]
            off = t * (_L * _UNROLL)
            for j in range(_UNROLL):
                m_, a_ = group_update(buf, off + j * _L, gi + j * _L,
                                      accs[2 * j], accs[2 * j + 1])
                accs[2 * j] = m_
                accs[2 * j + 1] = a_
            return (*accs, gi + _UNROLL * _L)

        big = jnp.full((_L,), _BIG, jnp.float32)
        zero_i = jnp.zeros((_L,), jnp.int32)
        init = (big, zero_i) * _UNROLL + (p0 + iota,)
        out = lax.fori_loop(0, _ITERS, body, init)
        accs = list(out[:-1])
        m, a = accs[0], accs[1]
        for j in range(1, _UNROLL):
            m, a = _merge(m, a, accs[2 * j], accs[2 * j + 1])

        mbuf[...] = m
        abuf[...] = a

        pltpu.sync_copy(mbuf, partm_hbm.at[c, s])
        pltpu.sync_copy(abuf, parta_hbm.at[c, s])
        plsc.subcore_barrier()

        @pl.when(s == 0)
        def _():
            pltpu.sync_copy(partm_hbm.at[c], allm_v)
            pltpu.sync_copy(parta_hbm.at[c], alli_v)
            gm = allm_v[0]
            ga = alli_v[0]
            for r in range(1, _NS):
                gm, ga = _merge(gm, ga, allm_v[r], alli_v[r])
            best = jnp.min(gm)
            cand = jnp.where(gm == jnp.full((_L,), best, jnp.float32),
                             ga, jnp.full((_L,), _IMAX, jnp.int32))
            besti = jnp.min(cand)
            mbuf[...] = jnp.full((_L,), best, jnp.float32)
            abuf[...] = jnp.full((_L,), besti, jnp.int32)
            pltpu.sync_copy(mbuf, outd_hbm.at[c])
            pltpu.sync_copy(abuf, outi_hbm.at[c])

            @pl.when(c == 0)
            def _():
                mbuf[...] = jnp.zeros((_L,), jnp.float32)
                pltpu.sync_copy(mbuf, ctrl_hbm)

    return k(path_t, svec)


def kernel(state, path):
    svec = jnp.broadcast_to(state[0, :_PATH_DIM][:, None], (3, _L))
    pad = jnp.full((_PATH_DIM, _N_PAD - _N_PATH), _BIG, jnp.float32)
    path_t = jnp.concatenate([path.T, pad], axis=1)
    outs = _sc_closest_point(path_t, svec)
    ctrl16 = outs[0]
    return ctrl16[:_CONTROL_DIM].reshape(1, _CONTROL_DIM)


# speedup vs baseline: 96.6532x; 1.3562x over previous
import functools

import jax
import jax.numpy as jnp
from jax import lax
from jax.experimental import pallas as pl
from jax.experimental.pallas import tpu as pltpu
from jax.experimental.pallas import tpu_sc as plsc

_N_PATH = 1_000_000
_PATH_DIM = 3
_CONTROL_DIM = 4

_NC = 2
_NS = 16
_NW = _NC * _NS
_L = 16

_PTS_W = 31_232
_GROUPS_W = _PTS_W // _L
_UNROLL = 8
_ITERS = _GROUPS_W // _UNROLL

_RES_START = _NW * _PTS_W
_RES_PTS = _N_PATH - _RES_START
_RES_PAD = 640

_BIG = float(jnp.finfo(jnp.float32).max)
_IMAX = 2**31 - 1


def _merge(m0, a0, m1, a1):
    take1 = (m1 < m0) | ((m1 == m0) & (a1 < a0))
    return jnp.where(take1, m1, m0), jnp.where(take1, a1, a0)


def _sc_closest_point(path_t, tail_pad, svec):
    mesh = plsc.VectorSubcoreMesh(core_axis_name="c", subcore_axis_name="s")

    @functools.partial(
        pl.kernel,
        mesh=mesh,
        compiler_params=pltpu.CompilerParams(needs_layout_passes=False),
        out_type=[
            jax.ShapeDtypeStruct((_L,), jnp.float32),
            jax.ShapeDtypeStruct((_NC, _L), jnp.float32),
            jax.ShapeDtypeStruct((_NC, _L), jnp.int32),
            jax.ShapeDtypeStruct((_NC, _NS, _L), jnp.float32),
            jax.ShapeDtypeStruct((_NC, _NS, _L), jnp.int32),
        ],
        scratch_types=[
            pltpu.VMEM((3, _PTS_W), jnp.float32),
            pltpu.VMEM((3, _L), jnp.float32),
            pltpu.VMEM((_L,), jnp.float32),
            pltpu.VMEM((_L,), jnp.int32),
            pltpu.VMEM((_NS, _L), jnp.float32),
            pltpu.VMEM((_NS, _L), jnp.int32),
            pltpu.SemaphoreType.DMA,
        ],
    )
    def k(path_hbm, tail_hbm, svec_hbm, ctrl_hbm, outd_hbm, outi_hbm,
          partm_hbm, parta_hbm,
          buf, svec_v, mbuf, abuf, allm_v, alli_v, sem):
        c = lax.axis_index("c")
        s = lax.axis_index("s")
        wid = s * _NC + c
        p0 = pl.multiple_of(wid * _PTS_W, 128)

        pltpu.sync_copy(svec_hbm, svec_v)
        pltpu.async_copy(path_hbm.at[:, pl.ds(p0, _PTS_W)], buf, sem).wait()

        sx = svec_v[0]
        sy = svec_v[1]
        sz = svec_v[2]
        iota = lax.iota(jnp.int32, _L)

        def group_update(b, off, gij, m, a):
            x = b[0, pl.ds(off, _L)]
            y = b[1, pl.ds(off, _L)]
            z = b[2, pl.ds(off, _L)]
            dx = x - sx
            dy = y - sy
            dz = z - sz
            d2 = dx * dx + dy * dy + dz * dz
            take = d2 < m
            return jnp.where(take, d2, m), jnp.where(take, gij, a)

        def body(t, carry):
            accs = list(carry[:-1])
            gi = carry[-1]
            off = t * (_L * _UNROLL)
            for j in range(_UNROLL):
                m_, a_ = group_update(buf, off + j * _L, gi + j * _L,
                                      accs[2 * j], accs[2 * j + 1])
                accs[2 * j] = m_
                accs[2 * j + 1] = a_
            return (*accs, gi + _UNROLL * _L)

        big = jnp.full((_L,), _BIG, jnp.float32)
        zero_i = jnp.zeros((_L,), jnp.int32)
        init = (big, zero_i) * _UNROLL + (p0 + iota,)
        out = lax.fori_loop(0, _ITERS, body, init)
        accs = list(out[:-1])
        m, a = accs[0], accs[1]
        for j in range(1, _UNROLL):
            m, a = _merge(m, a, accs[2 * j], accs[2 * j + 1])

        @pl.when(wid == _NW - 1)
        def _():
            pltpu.async_copy(
                tail_hbm, buf.at[:, pl.ds(0, _RES_PAD)], sem).wait()
            tm, ta = m, a
            for g in range(_RES_PAD // _L):
                tm, ta = group_update(buf, g * _L,
                                      _RES_START + g * _L + iota, tm, ta)
            mbuf[...] = tm
            abuf[...] = ta

        @pl.when(wid != _NW - 1)
        def _():
            mbuf[...] = m
            abuf[...] = a

        pltpu.sync_copy(mbuf, partm_hbm.at[c, s])
        pltpu.sync_copy(abuf, parta_hbm.at[c, s])
        plsc.subcore_barrier()

        @pl.when(s == 0)
        def _():
            pltpu.sync_copy(partm_hbm.at[c], allm_v)
            pltpu.sync_copy(parta_hbm.at[c], alli_v)
            gm = allm_v[0]
            ga = alli_v[0]
            for r in range(1, _NS):
                gm, ga = _merge(gm, ga, allm_v[r], alli_v[r])
            best = jnp.min(gm)
            cand = jnp.where(gm == jnp.full((_L,), best, jnp.float32),
                             ga, jnp.full((_L,), _IMAX, jnp.int32))
            besti = jnp.min(cand)
            mbuf[...] = jnp.full((_L,), best, jnp.float32)
            abuf[...] = jnp.full((_L,), besti, jnp.int32)
            pltpu.sync_copy(mbuf, outd_hbm.at[c])
            pltpu.sync_copy(abuf, outi_hbm.at[c])

            @pl.when(c == 0)
            def _():
                mbuf[...] = jnp.zeros((_L,), jnp.float32)
                pltpu.sync_copy(mbuf, ctrl_hbm)

    return k(path_t, tail_pad, svec)


def kernel(state, path):
    svec = jnp.broadcast_to(state[0, :_PATH_DIM][:, None], (3, _L))
    tail_pad = jnp.pad(path.T[:, _RES_START:], ((0, 0), (0, _RES_PAD - _RES_PTS)),
                       constant_values=_BIG)
    outs = _sc_closest_point(path.T, tail_pad, svec)
    ctrl16 = outs[0]
    return ctrl16[:_CONTROL_DIM].reshape(1, _CONTROL_DIM)


# speedup vs baseline: 100.4852x; 1.0396x over previous
import functools

import jax
import jax.numpy as jnp
from jax import lax
from jax.experimental import pallas as pl
from jax.experimental.pallas import tpu as pltpu
from jax.experimental.pallas import tpu_sc as plsc

_N_PATH = 1_000_000
_PATH_DIM = 3
_CONTROL_DIM = 4

_NC = 2
_NS = 16
_NW = _NC * _NS
_L = 16

_PTS_W = 31_232
_GROUPS_W = _PTS_W // _L
_UNROLL = 8
_ITERS = _GROUPS_W // _UNROLL

_RES_START = _NW * _PTS_W
_RES_PTS = _N_PATH - _RES_START
_RES_PAD = 640

_BIG = float(jnp.finfo(jnp.float32).max)
_IMAX = 2**31 - 1


def _merge(m0, a0, m1, a1):
    take1 = (m1 < m0) | ((m1 == m0) & (a1 < a0))
    return jnp.where(take1, m1, m0), jnp.where(take1, a1, a0)


def _sc_closest_point(path_t, tail_pad, svec):
    mesh = plsc.VectorSubcoreMesh(core_axis_name="c", subcore_axis_name="s")

    @functools.partial(
        pl.kernel,
        mesh=mesh,
        compiler_params=pltpu.CompilerParams(needs_layout_passes=False),
        out_type=[
            jax.ShapeDtypeStruct((_L,), jnp.float32),
            jax.ShapeDtypeStruct((_NC, _L), jnp.float32),
            jax.ShapeDtypeStruct((_NC, _L), jnp.int32),
            jax.ShapeDtypeStruct((_NC, _NS, _L), jnp.float32),
            jax.ShapeDtypeStruct((_NC, _NS, _L), jnp.int32),
        ],
        scratch_types=[
            pltpu.VMEM((3, _PTS_W), jnp.float32),
            pltpu.VMEM((3, _L), jnp.float32),
            pltpu.VMEM((_L,), jnp.float32),
            pltpu.VMEM((_L,), jnp.int32),
            pltpu.VMEM((_NS, _L), jnp.float32),
            pltpu.VMEM((_NS, _L), jnp.int32),
            pltpu.SemaphoreType.DMA,
            pltpu.SemaphoreType.DMA,
        ],
    )
    def k(path_hbm, tail_hbm, svec_hbm, ctrl_hbm, outd_hbm, outi_hbm,
          partm_hbm, parta_hbm,
          buf, svec_v, mbuf, abuf, allm_v, alli_v, sem, sem2):
        c = lax.axis_index("c")
        s = lax.axis_index("s")
        wid = s * _NC + c
        p0 = pl.multiple_of(wid * _PTS_W, 128)

        half = _PTS_W // 2
        dma1 = pltpu.async_copy(path_hbm.at[:, pl.ds(p0, half)],
                                buf.at[:, pl.ds(0, half)], sem)
        dma2 = pltpu.async_copy(path_hbm.at[:, pl.ds(p0 + half, half)],
                                buf.at[:, pl.ds(half, half)], sem2)
        pltpu.sync_copy(svec_hbm, svec_v)
        dma1.wait()

        sx = svec_v[0]
        sy = svec_v[1]
        sz = svec_v[2]
        iota = lax.iota(jnp.int32, _L)

        def group_update(b, off, gij, m, a):
            x = b[0, pl.ds(off, _L)]
            y = b[1, pl.ds(off, _L)]
            z = b[2, pl.ds(off, _L)]
            dx = x - sx
            dy = y - sy
            dz = z - sz
            d2 = dx * dx + dy * dy + dz * dz
            take = d2 < m
            return jnp.where(take, d2, m), jnp.where(take, gij, a)

        def body(t, carry):
            accs = list(carry[:-1])
            gi = carry[-1]
            off = t * (_L * _UNROLL)
            for j in range(_UNROLL):
                m_, a_ = group_update(buf, off + j * _L, gi + j * _L,
                                      accs[2 * j], accs[2 * j + 1])
                accs[2 * j] = m_
                accs[2 * j + 1] = a_
            return (*accs, gi + _UNROLL * _L)

        big = jnp.full((_L,), _BIG, jnp.float32)
        zero_i = jnp.zeros((_L,), jnp.int32)
        init = (big, zero_i) * _UNROLL + (p0 + iota,)
        mid = lax.fori_loop(0, _ITERS // 2, body, init)
        dma2.wait()
        out = lax.fori_loop(_ITERS // 2, _ITERS, body, mid)
        accs = list(out[:-1])
        m, a = accs[0], accs[1]
        for j in range(1, _UNROLL):
            m, a = _merge(m, a, accs[2 * j], accs[2 * j + 1])

        @pl.when(wid == _NW - 1)
        def _():
            pltpu.async_copy(
                tail_hbm, buf.at[:, pl.ds(0, _RES_PAD)], sem).wait()
            tm, ta = m, a
            for g in range(_RES_PAD // _L):
                tm, ta = group_update(buf, g * _L,
                                      _RES_START + g * _L + iota, tm, ta)
            mbuf[...] = tm
            abuf[...] = ta

        @pl.when(wid != _NW - 1)
        def _():
            mbuf[...] = m
            abuf[...] = a

        pltpu.sync_copy(mbuf, partm_hbm.at[c, s])
        pltpu.sync_copy(abuf, parta_hbm.at[c, s])
        plsc.subcore_barrier()

        @pl.when(s == 0)
        def _():
            pltpu.sync_copy(partm_hbm.at[c], allm_v)
            pltpu.sync_copy(parta_hbm.at[c], alli_v)
            gm = allm_v[0]
            ga = alli_v[0]
            for r in range(1, _NS):
                gm, ga = _merge(gm, ga, allm_v[r], alli_v[r])
            best = jnp.min(gm)
            cand = jnp.where(gm == jnp.full((_L,), best, jnp.float32),
                             ga, jnp.full((_L,), _IMAX, jnp.int32))
            besti = jnp.min(cand)
            mbuf[...] = jnp.full((_L,), best, jnp.float32)
            abuf[...] = jnp.full((_L,), besti, jnp.int32)
            pltpu.sync_copy(mbuf, outd_hbm.at[c])
            pltpu.sync_copy(abuf, outi_hbm.at[c])

            @pl.when(c == 0)
            def _():
                mbuf[...] = jnp.zeros((_L,), jnp.float32)
                pltpu.sync_copy(mbuf, ctrl_hbm)

    return k(path_t, tail_pad, svec)


def kernel(state, path):
    svec = jnp.broadcast_to(state[0, :_PATH_DIM][:, None], (3, _L))
    tail_pad = jnp.pad(path.T[:, _RES_START:], ((0, 0), (0, _RES_PAD - _RES_PTS)),
                       constant_values=_BIG)
    outs = _sc_closest_point(path.T, tail_pad, svec)
    ctrl16 = outs[0]
    return ctrl16[:_CONTROL_DIM].reshape(1, _CONTROL_DIM)
